# Initial kernel scaffold; baseline (speedup 1.0000x reference)
import jax
import jax.numpy as jnp
from jax.experimental import pallas as pl


def kernel(input_tensor, target):
    raise NotImplementedError("write your pallas kernel here")


# speedup vs baseline: 7.3693x; 7.3693x over previous
import functools

import jax
import jax.numpy as jnp
from jax import lax
from jax.experimental import pallas as pl
from jax.experimental.pallas import tpu as pltpu
from jax.experimental.pallas import tpu_sc as plsc

_B, _C, _H, _W = 8, 21, 512, 512
_N = _B * _H * _W
_K = max(1, int(0.7 * _N))


_BH = 64


def _ce_body(x_ref, t_ref, out_ref):
    t = t_ref[0]
    m = x_ref[0, 0]
    for c in range(1, _C):
        m = jnp.maximum(m, x_ref[0, c])
    s = jnp.zeros_like(m)
    xt = jnp.zeros_like(m)
    for c in range(_C):
        xc = x_ref[0, c]
        s = s + jnp.exp(xc - m)
        xt = jnp.where(t == c, xc, xt)
    out_ref[0] = (m + jnp.log(s)) - xt


def _ce_losses(x, t):
    return pl.pallas_call(
        _ce_body,
        grid=(_B, _H // _BH),
        in_specs=[
            pl.BlockSpec((1, _C, _BH, _W), lambda b, r: (b, 0, r, 0)),
            pl.BlockSpec((1, _BH, _W), lambda b, r: (b, r, 0)),
        ],
        out_specs=pl.BlockSpec((1, _BH, _W), lambda b, r: (b, r, 0)),
        out_shape=jax.ShapeDtypeStruct((_B, _H, _W), jnp.float32),
    )(x, t)


_NS = 16
_E = _N // _NS
_CHUNK = 4096
_NCHUNK = _E // _CHUNK
_NBIN = 2048
_HWORDS = _NS * _NBIN
_UNROLL = 8

_mesh = plsc.VectorSubcoreMesh(
    core_axis_name="c", subcore_axis_name="s", num_cores=1)


def _suffix_select(totals_ref, k, lane_iota):
    def body(j, carry):
        found, bsel, cnt_above, run = carry
        base = (_NBIN // 16 - 1 - j) * 16
        v = totals_ref[pl.ds(base, 16)]
        rev = lax.rev(v, (0,))
        cs = plsc.cumsum(rev)
        suffix = run + cs
        cond = suffix >= k
        any_now = jnp.any(cond)
        p = plsc.all_reduce_ffs(cond)
        sel = lane_iota == p
        cnt_geq = run + jnp.sum(jnp.where(sel, cs, 0))
        cnt_bin = jnp.sum(jnp.where(sel, rev, 0))
        bin_hit = jnp.sum(jnp.where(sel, base + 15 - lane_iota, 0))
        take = jnp.logical_and(any_now, found == 0)
        bsel = jnp.where(take, bin_hit, bsel)
        cnt_above = jnp.where(take, cnt_geq - cnt_bin, cnt_above)
        found = jnp.where(any_now, jnp.int32(1), found)
        run = run + jnp.sum(v)
        return found, bsel, cnt_above, run

    init = (jnp.int32(0), jnp.int32(0), jnp.int32(0), jnp.int32(0))
    _, bsel, cnt_above, _ = lax.fori_loop(0, _NBIN // 16, body, init)
    return bsel, cnt_above


@functools.partial(
    pl.kernel,
    out_type=jax.ShapeDtypeStruct((16,), jnp.float32),
    mesh=_mesh,
    compiler_params=pltpu.CompilerParams(needs_layout_passes=False),
    scratch_types=[
        pltpu.VMEM((_CHUNK,), jnp.float32),
        pltpu.VMEM((_HWORDS,), jnp.int32),
        pltpu.VMEM((_HWORDS,), jnp.int32),
        pltpu.VMEM((_NBIN,), jnp.int32),
        pltpu.VMEM((128,), jnp.int32),
        pltpu.VMEM((128,), jnp.int32),
        pltpu.VMEM((_NBIN,), jnp.int32),
        pltpu.VMEM((128,), jnp.float32),
        pltpu.VMEM((_NS * 128,), jnp.float32),
        pltpu.VMEM((16,), jnp.float32),
        pltpu.VMEM_SHARED((_NS, _NBIN), jnp.int32),
        pltpu.VMEM_SHARED((_NBIN,), jnp.int32),
        pltpu.VMEM_SHARED((_NS * 128,), jnp.float32),
    ],
)
def _select_kernel(loss_hbm, out_hbm, buf, hist1, hist2, lt, mtmp, macc,
                   totals, above_stage, above_all, out_stage,
                   sh_lt, sh_totals, sh_above):
    sid = lax.axis_index("s")
    lane_iota = lax.iota(jnp.int32, 16)
    lane_base = lane_iota * _NBIN
    zero16 = jnp.zeros((16,), jnp.int32)
    ones16 = jnp.ones((16,), jnp.int32)
    kk = jnp.int32(_K)

    def zero_ref(ref, nwords):
        def zbody(i, _):
            ref[pl.ds(i * 16, 16)] = zero16
            return 0
        lax.fori_loop(0, nwords // 16, zbody, 0)

    def lane_reduce(hist_ref):
        def rbody(i, _):
            ds = pl.ds(i * 16, 16)
            acc = hist_ref[ds]
            for lane in range(1, _NS):
                acc = acc + hist_ref[pl.ds(lane * _NBIN + i * 16, 16)]
            lt[ds] = acc
            return 0
        lax.fori_loop(0, _NBIN // 16, rbody, 0)

    def merge_slice():
        def mz(i, _):
            macc[pl.ds(i * 16, 16)] = zero16
            return 0
        lax.fori_loop(0, 8, mz, 0)
        for src in range(_NS):
            pltpu.sync_copy(sh_lt.at[src, pl.ds(sid * 128, 128)], mtmp)

            def madd(i, _):
                ds = pl.ds(i * 16, 16)
                macc[ds] = macc[ds] + mtmp[ds]
                return 0
            lax.fori_loop(0, 8, madd, 0)
        pltpu.sync_copy(macc, sh_totals.at[pl.ds(sid * 128, 128)])

    zero_ref(hist1, _HWORDS)
    zero_ref(hist2, _HWORDS)

    def chunk1(j, _):
        base = sid * _E + j * _CHUNK
        pltpu.sync_copy(loss_hbm.at[pl.ds(base, _CHUNK)], buf)

        def vec(i, _):
            for u in range(_UNROLL):
                v = buf[pl.ds((i * _UNROLL + u) * 16, 16)]
                bits = jnp.maximum(plsc.bitcast(v, jnp.int32), 0)
                b1 = lax.shift_right_logical(bits, 20)
                plsc.addupdate_scatter(hist1, [lane_base + b1], ones16)
            return 0

        lax.fori_loop(0, _CHUNK // 16 // _UNROLL, vec, 0)
        return 0

    lax.fori_loop(0, _NCHUNK, chunk1, 0)

    lane_reduce(hist1)
    pltpu.sync_copy(lt, sh_lt.at[sid])
    plsc.subcore_barrier()
    merge_slice()
    plsc.subcore_barrier()

    pltpu.sync_copy(sh_totals, totals)
    b1_sel, cnt_above1 = _suffix_select(totals, kk, lane_iota)
    k2 = kk - cnt_above1
    plsc.subcore_barrier()

    def chunk2(j, acc):
        base = sid * _E + j * _CHUNK
        pltpu.sync_copy(loss_hbm.at[pl.ds(base, _CHUNK)], buf)

        def vec(i, acc):
            for u in range(_UNROLL):
                v = buf[pl.ds((i * _UNROLL + u) * 16, 16)]
                bits = jnp.maximum(plsc.bitcast(v, jnp.int32), 0)
                b1 = lax.shift_right_logical(bits, 20)
                acc = acc + jnp.where(b1 > b1_sel, v, jnp.float32(0.0))
                sub = jnp.bitwise_and(lax.shift_right_logical(bits, 9),
                                      jnp.int32(0x7FF))
                plsc.addupdate_scatter(hist2, [lane_base + sub], ones16,
                                       mask=b1 == b1_sel)
            return acc

        return lax.fori_loop(0, _CHUNK // 16 // _UNROLL, vec, acc)

    above_acc = lax.fori_loop(0, _NCHUNK, chunk2,
                              jnp.zeros((16,), jnp.float32))
    zero16f = jnp.zeros((16,), jnp.float32)
    for q in range(8):
        above_stage[pl.ds(q * 16, 16)] = above_acc if q == 0 else zero16f
    pltpu.sync_copy(above_stage, sh_above.at[pl.ds(sid * 128, 128)])

    lane_reduce(hist2)
    pltpu.sync_copy(lt, sh_lt.at[sid])
    plsc.subcore_barrier()
    merge_slice()
    plsc.subcore_barrier()

    @pl.when(sid == 0)
    def _finalize():
        pltpu.sync_copy(sh_totals, totals)
        b2_sel, cnt_above2 = _suffix_select(totals, k2, lane_iota)

        def wbody(cidx, carry):
            wacc, c2acc = carry
            cnt = totals[pl.ds(cidx * 16, 16)]
            sbv = cidx * 16 + lane_iota
            center_bits = jnp.bitwise_or(
                jnp.bitwise_or(
                    lax.shift_left(b1_sel + zero16, 20),
                    lax.shift_left(sbv, 9)),
                jnp.int32(0x100))
            centers = plsc.bitcast(center_bits, jnp.float32)
            w = cnt.astype(jnp.float32) * centers
            wacc = wacc + jnp.where(sbv > b2_sel, w, jnp.float32(0.0))
            c2acc = c2acc + jnp.sum(
                jnp.where(sbv == b2_sel, centers, jnp.float32(0.0)))
            return wacc, c2acc

        wacc, center_b2 = lax.fori_loop(
            0, _NBIN // 16, wbody,
            (jnp.zeros((16,), jnp.float32), jnp.float32(0.0)))
        wsum = jnp.sum(wacc)

        pltpu.sync_copy(sh_above, above_all)

        def abody(i, acc):
            return acc + above_all[pl.ds(i * 16, 16)]

        asum_vec = lax.fori_loop(0, _NS * 8, abody,
                                 jnp.zeros((16,), jnp.float32))
        above_sum = jnp.sum(asum_vec)

        nsel = (k2 - cnt_above2).astype(jnp.float32)
        result = (above_sum + wsum + nsel * center_b2) * jnp.float32(1.0 / _K)
        out_stage[...] = jnp.zeros((16,), jnp.float32) + result
        pltpu.sync_copy(out_stage, out_hbm)


def kernel(input_tensor, target):
    losses = _ce_losses(input_tensor, target)
    out = _select_kernel(losses.reshape(-1))
    return out[0]


# speedup vs baseline: 19.3402x; 2.6244x over previous
import functools

import jax
import jax.numpy as jnp
from jax import lax
from jax.experimental import pallas as pl
from jax.experimental.pallas import tpu as pltpu
from jax.experimental.pallas import tpu_sc as plsc

_B, _C, _H, _W = 8, 21, 512, 512
_N = _B * _H * _W
_K = max(1, int(0.7 * _N))


_BH = 128


def _ce_body(x_ref, t_ref, out_ref):
    t = t_ref[0]
    m = x_ref[0, 0]
    for c in range(1, _C):
        m = jnp.maximum(m, x_ref[0, c])
    s = jnp.zeros_like(m)
    xt = jnp.zeros_like(m)
    for c in range(_C):
        xc = x_ref[0, c]
        s = s + jnp.exp(xc - m)
        xt = jnp.where(t == c, xc, xt)
    out_ref[0] = (m + jnp.log(s)) - xt


def _ce_losses(x, t):
    return pl.pallas_call(
        _ce_body,
        grid=(_B, _H // _BH),
        in_specs=[
            pl.BlockSpec((1, _C, _BH, _W), lambda b, r: (b, 0, r, 0)),
            pl.BlockSpec((1, _BH, _W), lambda b, r: (b, r, 0)),
        ],
        out_specs=pl.BlockSpec((1, _BH, _W), lambda b, r: (b, r, 0)),
        out_shape=jax.ShapeDtypeStruct((_B, _H, _W), jnp.float32),
    )(x, t)


_NS = 16
_NC = 2
_NW = _NS * _NC
_E = _N // _NW
_CHUNK = 8192
_NCHUNK = _E // _CHUNK
_VPC = _CHUNK // 16
_ROWS = _CHUNK // _W
_NBIN = 2048
_HCOPY = _NS * _NBIN
_HWORDS = 2 * _HCOPY

_mesh2 = plsc.VectorSubcoreMesh(
    core_axis_name="c", subcore_axis_name="s", num_cores=_NC)
_mesh1 = plsc.VectorSubcoreMesh(
    core_axis_name="c", subcore_axis_name="s", num_cores=1)
_cparams = pltpu.CompilerParams(needs_layout_passes=False)


def _suffix_select(totals_ref, k, lane_iota):
    def body(j, carry):
        found, bsel, cnt_above, run = carry
        base = (_NBIN // 16 - 1 - j) * 16
        v = totals_ref[pl.ds(base, 16)]
        rev = lax.rev(v, (0,))
        cs = plsc.cumsum(rev)
        suffix = run + cs
        cond = suffix >= k
        any_now = jnp.any(cond)
        p = plsc.all_reduce_ffs(cond)
        sel = lane_iota == p
        cnt_geq = run + jnp.sum(jnp.where(sel, cs, 0))
        cnt_bin = jnp.sum(jnp.where(sel, rev, 0))
        bin_hit = jnp.sum(jnp.where(sel, base + 15 - lane_iota, 0))
        take = jnp.logical_and(any_now, found == 0)
        bsel = jnp.where(take, bin_hit, bsel)
        cnt_above = jnp.where(take, cnt_geq - cnt_bin, cnt_above)
        found = jnp.where(any_now, jnp.int32(1), found)
        run = run + jnp.sum(v)
        return found, bsel, cnt_above, run

    init = (jnp.int32(0), jnp.int32(0), jnp.int32(0), jnp.int32(0))
    _, bsel, cnt_above, _ = lax.fori_loop(0, _NBIN // 16, body, init)
    return bsel, cnt_above


_STREAM_SCRATCH = [
    pltpu.VMEM((_ROWS, _W), jnp.float32),
    pltpu.VMEM((_ROWS, _W), jnp.float32),
    pltpu.VMEM((_HWORDS,), jnp.int32),
    pltpu.VMEM((_NBIN,), jnp.int32),
    pltpu.VMEM((_NS, 128), jnp.int32),
    pltpu.VMEM((128,), jnp.int32),
    pltpu.SemaphoreType.DMA,
    pltpu.SemaphoreType.DMA,
    pltpu.VMEM_SHARED((_NS, _NBIN), jnp.int32),
]


def _tile_helpers(hist, lt, mgather, macc, sh_lt, sid):
    lane_iota = lax.iota(jnp.int32, 16)
    lane_base = lane_iota * _NBIN
    zero16 = jnp.zeros((16,), jnp.int32)

    def zero_hist():
        @plsc.parallel_loop(0, _HWORDS // 16, unroll=8)
        def _(i):
            hist[pl.ds(i * 16, 16)] = zero16

    def lane_reduce():
        @plsc.parallel_loop(0, _NBIN // 16, unroll=2)
        def _(i):
            ds = pl.ds(i * 16, 16)
            acc = hist[ds]
            for blk in range(1, 2 * _NS):
                acc = acc + hist[pl.ds(blk * _NBIN + i * 16, 16)]
            lt[ds] = acc

    def merge_slice(dst_hbm_slice):
        pltpu.sync_copy(lt, sh_lt.at[sid])
        plsc.subcore_barrier()
        pltpu.sync_copy(sh_lt.at[:, pl.ds(sid * 128, 128)], mgather)

        @plsc.parallel_loop(0, 8, unroll=8)
        def _(q):
            ds = pl.ds(q * 16, 16)
            acc = mgather[0, ds]
            for src in range(1, _NS):
                acc = acc + mgather[src, ds]
            macc[ds] = acc

        pltpu.sync_copy(macc, dst_hbm_slice)

    return lane_iota, lane_base, zero_hist, lane_reduce, merge_slice


def _stream_loop(loss_hbm, bufs, sems, row0, process):
    descs = []
    for j in range(_NCHUNK):
        src = loss_hbm.at[pl.ds(row0 + j * _ROWS, _ROWS), :]
        descs.append(pltpu.make_async_copy(src, bufs[j % 2], sems[j % 2]))
    descs[0].start()
    for j in range(_NCHUNK):
        descs[j].wait()
        if j + 1 < _NCHUNK:
            descs[j + 1].start()
        process(bufs[j % 2])


@functools.partial(
    pl.kernel,
    out_type=jax.ShapeDtypeStruct((_NC, _NBIN), jnp.int32),
    mesh=_mesh2,
    compiler_params=_cparams,
    scratch_types=_STREAM_SCRATCH,
)
def _pass1_kernel(loss_hbm, h1_hbm, buf0, buf1, hist, lt, mgather, macc,
                  sem0, sem1, sh_lt):
    cid = lax.axis_index("c")
    sid = lax.axis_index("s")
    wid = cid * _NS + sid
    row0 = wid * (_E // _W)
    (lane_iota, lane_base, zero_hist, lane_reduce,
     merge_slice) = _tile_helpers(hist, lt, mgather, macc, sh_lt, sid)
    ones16 = jnp.ones((16,), jnp.int32)

    zero_hist()

    def p1_process(buf):
        @plsc.parallel_loop(0, _VPC, unroll=8)
        def _(i):
            v = buf[i >> 5, pl.ds(jnp.bitwise_and(i, 31) * 16, 16)]
            bits = jnp.maximum(plsc.bitcast(v, jnp.int32), 0)
            b1 = lax.shift_right_logical(bits, 20)
            par = jnp.bitwise_and(i, 1) * _HCOPY
            plsc.addupdate_scatter(hist, [par + lane_base + b1], ones16)

    _stream_loop(loss_hbm, (buf0, buf1), (sem0, sem1), row0, p1_process)
    lane_reduce()
    merge_slice(h1_hbm.at[cid, pl.ds(sid * 128, 128)])


@functools.partial(
    pl.kernel,
    out_type=(jax.ShapeDtypeStruct((_NC, _NBIN), jnp.int32),
              jax.ShapeDtypeStruct((_NC, _NBIN), jnp.float32)),
    mesh=_mesh2,
    compiler_params=_cparams,
    scratch_types=_STREAM_SCRATCH + [
        pltpu.VMEM((_NC, _NBIN), jnp.int32),
        pltpu.VMEM((_NBIN,), jnp.int32),
        pltpu.VMEM((128,), jnp.float32),
    ],
)
def _pass2_kernel(loss_hbm, h1_hbm, sub_hbm, above_hbm, buf0, buf1, hist, lt,
                  mgather, macc, sem0, sem1, sh_lt, h1tmp, totals,
                  above_stage):
    cid = lax.axis_index("c")
    sid = lax.axis_index("s")
    wid = cid * _NS + sid
    row0 = wid * (_E // _W)
    (lane_iota, lane_base, zero_hist, lane_reduce,
     merge_slice) = _tile_helpers(hist, lt, mgather, macc, sh_lt, sid)
    ones16 = jnp.ones((16,), jnp.int32)
    kk = jnp.int32(_K)

    pltpu.sync_copy(h1_hbm, h1tmp)

    @plsc.parallel_loop(0, _NBIN // 16, unroll=8)
    def _(i):
        ds = pl.ds(i * 16, 16)
        totals[ds] = h1tmp[0, ds] + h1tmp[1, ds]

    b1_sel, cnt_above1 = _suffix_select(totals, kk, lane_iota)

    zero_hist()
    zero16f = jnp.zeros((16,), jnp.float32)
    acc_pair = [(zero16f, zero16f)]

    def p2_process(buf):
        @plsc.parallel_loop(0, _VPC, unroll=8, carry=acc_pair[0])
        def _(i, carry):
            a0, a1 = carry
            v = buf[i >> 5, pl.ds(jnp.bitwise_and(i, 31) * 16, 16)]
            bits = jnp.maximum(plsc.bitcast(v, jnp.int32), 0)
            b1 = lax.shift_right_logical(bits, 20)
            sub = jnp.bitwise_and(lax.shift_right_logical(bits, 9),
                                  jnp.int32(0x7FF))
            par = jnp.bitwise_and(i, 1) * _HCOPY
            plsc.addupdate_scatter(hist, [par + lane_base + sub], ones16,
                                   mask=b1 == b1_sel)
            return a1, a0 + jnp.where(b1 > b1_sel, v, jnp.float32(0.0))

        acc_pair[0] = _

    _stream_loop(loss_hbm, (buf0, buf1), (sem0, sem1), row0, p2_process)

    above_acc = acc_pair[0][0] + acc_pair[0][1]
    for q in range(8):
        above_stage[pl.ds(q * 16, 16)] = above_acc if q == 0 else zero16f
    pltpu.sync_copy(above_stage,
                    above_hbm.at[cid, pl.ds(sid * 128, 128)])
    lane_reduce()
    merge_slice(sub_hbm.at[cid, pl.ds(sid * 128, 128)])


@functools.partial(
    pl.kernel,
    out_type=jax.ShapeDtypeStruct((16,), jnp.float32),
    mesh=_mesh1,
    compiler_params=_cparams,
    scratch_types=[
        pltpu.VMEM((_NC, _NBIN), jnp.int32),
        pltpu.VMEM((_NBIN,), jnp.int32),
        pltpu.VMEM((_NC, _NBIN), jnp.float32),
        pltpu.VMEM((16,), jnp.float32),
    ],
)
def _final_kernel(h1_hbm, sub_hbm, above_hbm, out_hbm, htmp, totals,
                  abovetmp, out_stage):
    sid = lax.axis_index("s")
    lane_iota = lax.iota(jnp.int32, 16)
    zero16 = jnp.zeros((16,), jnp.int32)
    kk = jnp.int32(_K)

    @pl.when(jnp.logical_and(sid == 0, lax.axis_index("c") == 0))
    def _finalize():
        pltpu.sync_copy(h1_hbm, htmp)

        def mbody(i, _):
            ds = pl.ds(i * 16, 16)
            totals[ds] = htmp[0, ds] + htmp[1, ds]
            return 0

        lax.fori_loop(0, _NBIN // 16, mbody, 0)
        b1_sel, cnt_above1 = _suffix_select(totals, kk, lane_iota)
        k2 = kk - cnt_above1

        pltpu.sync_copy(sub_hbm, htmp)
        lax.fori_loop(0, _NBIN // 16, mbody, 0)
        b2_sel, cnt_above2 = _suffix_select(totals, k2, lane_iota)

        def wbody(cidx, carry):
            wacc, c2acc = carry
            cnt = totals[pl.ds(cidx * 16, 16)]
            sbv = cidx * 16 + lane_iota
            center_bits = jnp.bitwise_or(
                jnp.bitwise_or(
                    lax.shift_left(b1_sel + zero16, 20),
                    lax.shift_left(sbv, 9)),
                jnp.int32(0x100))
            centers = plsc.bitcast(center_bits, jnp.float32)
            w = cnt.astype(jnp.float32) * centers
            wacc = wacc + jnp.where(sbv > b2_sel, w, jnp.float32(0.0))
            c2acc = c2acc + jnp.sum(
                jnp.where(sbv == b2_sel, centers, jnp.float32(0.0)))
            return wacc, c2acc

        wacc, center_b2 = lax.fori_loop(
            0, _NBIN // 16, wbody,
            (jnp.zeros((16,), jnp.float32), jnp.float32(0.0)))
        wsum = jnp.sum(wacc)

        pltpu.sync_copy(above_hbm, abovetmp)

        def abody(i, acc):
            return (acc + abovetmp[0, pl.ds(i * 16, 16)]
                    + abovetmp[1, pl.ds(i * 16, 16)])

        asum_vec = lax.fori_loop(0, _NBIN // 16, abody,
                                 jnp.zeros((16,), jnp.float32))
        above_sum = jnp.sum(asum_vec)

        nsel = (k2 - cnt_above2).astype(jnp.float32)
        result = (above_sum + wsum + nsel * center_b2) * jnp.float32(1.0 / _K)
        out_stage[...] = jnp.zeros((16,), jnp.float32) + result
        pltpu.sync_copy(out_stage, out_hbm)


def kernel(input_tensor, target):
    losses = _ce_losses(input_tensor, target)
    flat = losses.reshape(_B * _H, _W)
    h1 = _pass1_kernel(flat)
    sub, above = _pass2_kernel(flat, h1)
    out = _final_kernel(h1, sub, above)
    return out[0]


# speedup vs baseline: 20.3781x; 1.0537x over previous
import functools

import jax
import jax.numpy as jnp
from jax import lax
from jax.experimental import pallas as pl
from jax.experimental.pallas import tpu as pltpu
from jax.experimental.pallas import tpu_sc as plsc

_B, _C, _H, _W = 8, 21, 512, 512
_N = _B * _H * _W
_K = max(1, int(0.7 * _N))


_BH = 256


def _ce_body(x_ref, t_ref, out_ref):
    t = t_ref[0]
    m = x_ref[0, 0]
    for c in range(1, _C):
        m = jnp.maximum(m, x_ref[0, c])
    s = jnp.zeros_like(m)
    xt = jnp.zeros_like(m)
    for c in range(_C):
        xc = x_ref[0, c]
        s = s + jnp.exp(xc - m)
        xt = jnp.where(t == c, xc, xt)
    out_ref[0] = (m + jnp.log(s)) - xt


def _ce_losses(x, t):
    return pl.pallas_call(
        _ce_body,
        grid=(_B, _H // _BH),
        in_specs=[
            pl.BlockSpec((1, _C, _BH, _W), lambda b, r: (b, 0, r, 0)),
            pl.BlockSpec((1, _BH, _W), lambda b, r: (b, r, 0)),
        ],
        out_specs=pl.BlockSpec((1, _BH, _W), lambda b, r: (b, r, 0)),
        out_shape=jax.ShapeDtypeStruct((_B, _H, _W), jnp.float32),
    )(x, t)


_NS = 16
_NC = 2
_NW = _NS * _NC
_E = _N // _NW
_CHUNK = 8192
_NCHUNK = _E // _CHUNK
_VPC = _CHUNK // 16
_ROWS = _CHUNK // _W
_NBIN = 2048
_HCOPY = _NS * _NBIN
_HWORDS = 2 * _HCOPY

_mesh2 = plsc.VectorSubcoreMesh(
    core_axis_name="c", subcore_axis_name="s", num_cores=_NC)
_mesh1 = plsc.VectorSubcoreMesh(
    core_axis_name="c", subcore_axis_name="s", num_cores=1)
_cparams = pltpu.CompilerParams(needs_layout_passes=False)


def _suffix_select(totals_ref, k, lane_iota):
    def body(j, carry):
        found, bsel, cnt_above, run = carry
        base = (_NBIN // 16 - 1 - j) * 16
        v = totals_ref[pl.ds(base, 16)]
        rev = lax.rev(v, (0,))
        cs = plsc.cumsum(rev)
        suffix = run + cs
        cond = suffix >= k
        any_now = jnp.any(cond)
        p = plsc.all_reduce_ffs(cond)
        sel = lane_iota == p
        cnt_geq = run + jnp.sum(jnp.where(sel, cs, 0))
        cnt_bin = jnp.sum(jnp.where(sel, rev, 0))
        bin_hit = jnp.sum(jnp.where(sel, base + 15 - lane_iota, 0))
        take = jnp.logical_and(any_now, found == 0)
        bsel = jnp.where(take, bin_hit, bsel)
        cnt_above = jnp.where(take, cnt_geq - cnt_bin, cnt_above)
        found = jnp.where(any_now, jnp.int32(1), found)
        run = run + jnp.sum(v)
        return found, bsel, cnt_above, run

    init = (jnp.int32(0), jnp.int32(0), jnp.int32(0), jnp.int32(0))
    _, bsel, cnt_above, _ = lax.fori_loop(0, _NBIN // 16, body, init)
    return bsel, cnt_above


_STREAM_SCRATCH = [
    pltpu.VMEM((_ROWS, _W), jnp.float32),
    pltpu.VMEM((_ROWS, _W), jnp.float32),
    pltpu.VMEM((_HWORDS,), jnp.int32),
    pltpu.VMEM((_NBIN,), jnp.int32),
    pltpu.VMEM((_NS, 128), jnp.int32),
    pltpu.VMEM((128,), jnp.int32),
    pltpu.SemaphoreType.DMA,
    pltpu.SemaphoreType.DMA,
    pltpu.VMEM_SHARED((_NS, _NBIN), jnp.int32),
]


def _tile_helpers(hist, lt, mgather, macc, sh_lt, sid):
    lane_iota = lax.iota(jnp.int32, 16)
    lane_base = lane_iota * _NBIN
    zero16 = jnp.zeros((16,), jnp.int32)

    def zero_hist():
        @plsc.parallel_loop(0, _HWORDS // 16, unroll=8)
        def _(i):
            hist[pl.ds(i * 16, 16)] = zero16

    def lane_reduce():
        @plsc.parallel_loop(0, _NBIN // 16, unroll=2)
        def _(i):
            ds = pl.ds(i * 16, 16)
            acc = hist[ds]
            for blk in range(1, 2 * _NS):
                acc = acc + hist[pl.ds(blk * _NBIN + i * 16, 16)]
            lt[ds] = acc

    def merge_slice(dst_hbm_slice):
        pltpu.sync_copy(lt, sh_lt.at[sid])
        plsc.subcore_barrier()
        pltpu.sync_copy(sh_lt.at[:, pl.ds(sid * 128, 128)], mgather)

        @plsc.parallel_loop(0, 8, unroll=8)
        def _(q):
            ds = pl.ds(q * 16, 16)
            acc = mgather[0, ds]
            for src in range(1, _NS):
                acc = acc + mgather[src, ds]
            macc[ds] = acc

        pltpu.sync_copy(macc, dst_hbm_slice)

    return lane_iota, lane_base, zero_hist, lane_reduce, merge_slice


def _stream_loop(loss_hbm, bufs, sems, row0, process):
    descs = []
    for j in range(_NCHUNK):
        src = loss_hbm.at[pl.ds(row0 + j * _ROWS, _ROWS), :]
        descs.append(pltpu.make_async_copy(src, bufs[j % 2], sems[j % 2]))
    descs[0].start()
    for j in range(_NCHUNK):
        descs[j].wait()
        if j + 1 < _NCHUNK:
            descs[j + 1].start()
        process(bufs[j % 2])


@functools.partial(
    pl.kernel,
    out_type=jax.ShapeDtypeStruct((_NC, _NBIN), jnp.int32),
    mesh=_mesh2,
    compiler_params=_cparams,
    scratch_types=_STREAM_SCRATCH,
)
def _pass1_kernel(loss_hbm, h1_hbm, buf0, buf1, hist, lt, mgather, macc,
                  sem0, sem1, sh_lt):
    cid = lax.axis_index("c")
    sid = lax.axis_index("s")
    wid = cid * _NS + sid
    row0 = wid * (_E // _W)
    (lane_iota, lane_base, zero_hist, lane_reduce,
     merge_slice) = _tile_helpers(hist, lt, mgather, macc, sh_lt, sid)
    ones16 = jnp.ones((16,), jnp.int32)

    zero_hist()

    def p1_process(buf):
        @plsc.parallel_loop(0, _VPC, unroll=8)
        def _(i):
            v = buf[i >> 5, pl.ds(jnp.bitwise_and(i, 31) * 16, 16)]
            bits = jnp.maximum(plsc.bitcast(v, jnp.int32), 0)
            b1 = lax.shift_right_logical(bits, 20)
            par = jnp.bitwise_and(i, 1) * _HCOPY
            plsc.addupdate_scatter(hist, [par + lane_base + b1], ones16)

    _stream_loop(loss_hbm, (buf0, buf1), (sem0, sem1), row0, p1_process)
    lane_reduce()
    merge_slice(h1_hbm.at[cid, pl.ds(sid * 128, 128)])


@functools.partial(
    pl.kernel,
    out_type=(jax.ShapeDtypeStruct((_NC, _NBIN), jnp.int32),
              jax.ShapeDtypeStruct((_NC, _NBIN), jnp.float32)),
    mesh=_mesh2,
    compiler_params=_cparams,
    scratch_types=_STREAM_SCRATCH + [
        pltpu.VMEM((_NC, _NBIN), jnp.int32),
        pltpu.VMEM((_NBIN,), jnp.int32),
        pltpu.VMEM((128,), jnp.float32),
    ],
)
def _pass2_kernel(loss_hbm, h1_hbm, sub_hbm, above_hbm, buf0, buf1, hist, lt,
                  mgather, macc, sem0, sem1, sh_lt, h1tmp, totals,
                  above_stage):
    cid = lax.axis_index("c")
    sid = lax.axis_index("s")
    wid = cid * _NS + sid
    row0 = wid * (_E // _W)
    (lane_iota, lane_base, zero_hist, lane_reduce,
     merge_slice) = _tile_helpers(hist, lt, mgather, macc, sh_lt, sid)
    ones16 = jnp.ones((16,), jnp.int32)
    kk = jnp.int32(_K)

    pltpu.sync_copy(h1_hbm, h1tmp)

    @plsc.parallel_loop(0, _NBIN // 16, unroll=8)
    def _(i):
        ds = pl.ds(i * 16, 16)
        totals[ds] = h1tmp[0, ds] + h1tmp[1, ds]

    b1_sel, cnt_above1 = _suffix_select(totals, kk, lane_iota)

    zero_hist()
    zero16f = jnp.zeros((16,), jnp.float32)
    acc_pair = [(zero16f, zero16f)]

    def p2_process(buf):
        @plsc.parallel_loop(0, _VPC, unroll=8, carry=acc_pair[0])
        def _(i, carry):
            a0, a1 = carry
            v = buf[i >> 5, pl.ds(jnp.bitwise_and(i, 31) * 16, 16)]
            bits = jnp.maximum(plsc.bitcast(v, jnp.int32), 0)
            b1 = lax.shift_right_logical(bits, 20)
            sub = jnp.bitwise_and(lax.shift_right_logical(bits, 9),
                                  jnp.int32(0x7FF))
            par = jnp.bitwise_and(i, 1) * _HCOPY
            plsc.addupdate_scatter(hist, [par + lane_base + sub], ones16,
                                   mask=b1 == b1_sel)
            return a1, a0 + jnp.where(b1 > b1_sel, v, jnp.float32(0.0))

        acc_pair[0] = _

    _stream_loop(loss_hbm, (buf0, buf1), (sem0, sem1), row0, p2_process)

    above_acc = acc_pair[0][0] + acc_pair[0][1]
    for q in range(8):
        above_stage[pl.ds(q * 16, 16)] = above_acc if q == 0 else zero16f
    pltpu.sync_copy(above_stage,
                    above_hbm.at[cid, pl.ds(sid * 128, 128)])
    lane_reduce()
    merge_slice(sub_hbm.at[cid, pl.ds(sid * 128, 128)])


@functools.partial(
    pl.kernel,
    out_type=jax.ShapeDtypeStruct((16,), jnp.float32),
    mesh=_mesh1,
    compiler_params=_cparams,
    scratch_types=[
        pltpu.VMEM((_NC, _NBIN), jnp.int32),
        pltpu.VMEM((_NBIN,), jnp.int32),
        pltpu.VMEM((_NC, _NBIN), jnp.float32),
        pltpu.VMEM((16,), jnp.float32),
    ],
)
def _final_kernel(h1_hbm, sub_hbm, above_hbm, out_hbm, htmp, totals,
                  abovetmp, out_stage):
    sid = lax.axis_index("s")
    lane_iota = lax.iota(jnp.int32, 16)
    zero16 = jnp.zeros((16,), jnp.int32)
    kk = jnp.int32(_K)

    @pl.when(jnp.logical_and(sid == 0, lax.axis_index("c") == 0))
    def _finalize():
        pltpu.sync_copy(h1_hbm, htmp)

        def mbody(i, _):
            ds = pl.ds(i * 16, 16)
            totals[ds] = htmp[0, ds] + htmp[1, ds]
            return 0

        lax.fori_loop(0, _NBIN // 16, mbody, 0)
        b1_sel, cnt_above1 = _suffix_select(totals, kk, lane_iota)
        k2 = kk - cnt_above1

        pltpu.sync_copy(sub_hbm, htmp)
        lax.fori_loop(0, _NBIN // 16, mbody, 0)
        b2_sel, cnt_above2 = _suffix_select(totals, k2, lane_iota)

        def wbody(cidx, carry):
            wacc, c2acc = carry
            cnt = totals[pl.ds(cidx * 16, 16)]
            sbv = cidx * 16 + lane_iota
            center_bits = jnp.bitwise_or(
                jnp.bitwise_or(
                    lax.shift_left(b1_sel + zero16, 20),
                    lax.shift_left(sbv, 9)),
                jnp.int32(0x100))
            centers = plsc.bitcast(center_bits, jnp.float32)
            w = cnt.astype(jnp.float32) * centers
            wacc = wacc + jnp.where(sbv > b2_sel, w, jnp.float32(0.0))
            c2acc = c2acc + jnp.sum(
                jnp.where(sbv == b2_sel, centers, jnp.float32(0.0)))
            return wacc, c2acc

        wacc, center_b2 = lax.fori_loop(
            0, _NBIN // 16, wbody,
            (jnp.zeros((16,), jnp.float32), jnp.float32(0.0)))
        wsum = jnp.sum(wacc)

        pltpu.sync_copy(above_hbm, abovetmp)

        def abody(i, acc):
            return (acc + abovetmp[0, pl.ds(i * 16, 16)]
                    + abovetmp[1, pl.ds(i * 16, 16)])

        asum_vec = lax.fori_loop(0, _NBIN // 16, abody,
                                 jnp.zeros((16,), jnp.float32))
        above_sum = jnp.sum(asum_vec)

        nsel = (k2 - cnt_above2).astype(jnp.float32)
        result = (above_sum + wsum + nsel * center_b2) * jnp.float32(1.0 / _K)
        out_stage[...] = jnp.zeros((16,), jnp.float32) + result
        pltpu.sync_copy(out_stage, out_hbm)


def kernel(input_tensor, target):
    losses = _ce_losses(input_tensor, target)
    flat = losses.reshape(_B * _H, _W)
    h1 = _pass1_kernel(flat)
    sub, above = _pass2_kernel(flat, h1)
    out = _final_kernel(h1, sub, above)
    return out[0]
